# baseline (speedup 1.0000x reference)
import dataclasses
import functools

import jax
import jax.numpy as jnp
from jax import lax
from jax.experimental import pallas as pl
from jax.experimental.pallas import tpu as pltpu
from jax.experimental.pallas import tpu_sc as plsc

N_NODES = 10000
K_NEIGH = 32
D_FEAT = 128
SUBDIV = 3
SCALER = 10.0 * 1.0 * float(SUBDIV)
D_OUT = SUBDIV * D_FEAT
W_COLS = SUBDIV * K_NEIGH

NUM_CORES = 2
NUM_SUBCORES = 16
NUM_WORKERS = NUM_CORES * NUM_SUBCORES

VPW = 320
N_PAD = NUM_WORKERS * VPW
OCT = 8
NOCT = VPW // OCT
OCT_ROWS = OCT * K_NEIGH

LANES = 16
FVECS = D_FEAT // LANES


def _aux_body(d_ref, o_ref):
    dist = jnp.sqrt(d_ref[...] + 1e-6)
    ws = []
    for i in range(SUBDIV):
        offset = float(i) / float(SUBDIV)
        w = jnp.exp(-SCALER * (dist - offset) ** 2)
        wsum = jnp.sum(w, axis=1, keepdims=True) + 1e-6
        ws.append(w / wsum)
    o_ref[...] = jnp.concatenate(ws, axis=-1)


def _aux_weights(distsq):
    return pl.pallas_call(
        _aux_body,
        grid=(10,),
        in_specs=[pl.BlockSpec((N_NODES // 10, K_NEIGH), lambda b: (b, 0))],
        out_specs=pl.BlockSpec((N_NODES // 10, W_COLS), lambda b: (b, 0)),
        out_shape=jax.ShapeDtypeStruct((N_NODES, W_COLS), jnp.float32),
    )(distsq)


def _sc_compiler_params():
    cp = pltpu.CompilerParams()
    if "needs_layout_passes" in pltpu.CompilerParams.__dataclass_fields__:
        cp = dataclasses.replace(cp, needs_layout_passes=False)
    return cp


def _sc_fused(features, idx_pad, aux_pad):
    mesh = plsc.VectorSubcoreMesh(core_axis_name="c", subcore_axis_name="s")

    @functools.partial(
        pl.kernel,
        compiler_params=_sc_compiler_params(),
        out_type=jax.ShapeDtypeStruct((N_NODES, D_OUT), jnp.float32),
        mesh=mesh,
        scratch_types=[
            pltpu.VMEM((8, 128), jnp.int32),
            pltpu.VMEM((2, OCT, W_COLS), jnp.float32),
            pltpu.VMEM((2, OCT_ROWS, D_FEAT), jnp.float32),
            pltpu.VMEM((2, OCT, D_OUT), jnp.float32),
            pltpu.SemaphoreType.DMA((2,)),
            pltpu.SemaphoreType.DMA((2,)),
            pltpu.SemaphoreType.DMA((2,)),
            pltpu.SemaphoreType.DMA((2,)),
        ],
    )
    def fused_kernel(
        feat_hbm, idxf_hbm, aux_hbm, out_hbm,
        idx_b, aux_b, rows_v, out_b, isem, asem, gsem, osem,
    ):
        wid = lax.axis_index("s") * NUM_CORES + lax.axis_index("c")
        base = wid * VPW

        def idx_load_super(s):
            row = pl.multiple_of((base + s * 4 * OCT) * K_NEIGH // 128, 8)
            pltpu.sync_copy(idxf_hbm.at[pl.ds(row, 8)], idx_b)

        def aux_copy(ob, o):
            return pltpu.make_async_copy(
                aux_hbm.at[pl.ds(pl.multiple_of(base + o * OCT, OCT), OCT)],
                aux_b.at[ob],
                asem.at[ob],
            )

        def gathers(b, om4):
            return [
                pltpu.make_async_copy(
                    feat_hbm.at[idx_b.at[om4 * 2 + h]],
                    rows_v.at[b].at[pl.ds(h * 128, 128)],
                    gsem.at[b],
                )
                for h in range(2)
            ]

        def gather_start(b, om4):
            for c in gathers(b, om4):
                c.start()

        def gather_wait(b, om4):
            for c in gathers(b, om4):
                c.wait()

        def out_copy(ob, gv0):
            return pltpu.make_async_copy(
                out_b.at[ob],
                out_hbm.at[pl.ds(pl.multiple_of(gv0, OCT), OCT)],
                osem.at[ob],
            )

        def compute(ob, p):
            zero = jnp.zeros((LANES,), jnp.float32)
            half_n = FVECS // 2
            accs_out = [[None] * FVECS for _ in range(SUBDIV)]
            p_vec = jnp.full((LANES,), p, jnp.int32)
            for half in range(2):
                init = tuple(zero for _ in range(SUBDIV * half_n))

                def body(k, accs, half=half):
                    accs = list(accs)
                    wv = [
                        plsc.load_gather(
                            aux_b.at[ob],
                            [
                                p_vec,
                                jnp.full((LANES,), i * K_NEIGH, jnp.int32) + k,
                            ],
                        )
                        for i in range(SUBDIV)
                    ]
                    for fi in range(half_n):
                        fv = half * half_n + fi
                        g = rows_v[ob, p * K_NEIGH + k, pl.ds(fv * LANES, LANES)]
                        for i in range(SUBDIV):
                            accs[i * half_n + fi] = (
                                accs[i * half_n + fi] + wv[i] * g
                            )
                    return tuple(accs)

                accs = lax.fori_loop(0, K_NEIGH, body, init, unroll=4)
                for i in range(SUBDIV):
                    for fi in range(half_n):
                        accs_out[i][half * half_n + fi] = accs[i * half_n + fi]
            return accs_out

        idx_load_super(0)
        for ob in range(2):
            aux_copy(ob, ob).start()
            gather_start(ob, ob)

        @pl.loop(0, NOCT // 4)
        def _(n4):
            for ob in range(4):
                b = ob % 2
                o = n4 * 4 + ob
                gv0 = base + o * OCT

                if ob != 3:
                    gather_wait(b, ob)
                aux_copy(b, o).wait()

                @pl.when((o >= 2) & (gv0 - 2 * OCT < N_NODES))
                def _(b=b):
                    out_copy(b, base).wait()

                for p in range(OCT):
                    accs = compute(b, p)
                    for i in range(SUBDIV):
                        for fv in range(FVECS):
                            out_b[
                                b, p, pl.ds(i * D_FEAT + fv * LANES, LANES)
                            ] = accs[i][fv]

                @pl.when(gv0 < N_NODES)
                def _(b=b, gv0=gv0):
                    out_copy(b, gv0).start()

                @pl.when(o + 2 < NOCT)
                def _(b=b, o=o):
                    aux_copy(b, o + 2).start()

                if ob == 2:
                    gather_wait(1, 3)

                    @pl.when(n4 + 1 < NOCT // 4)
                    def _():
                        idx_load_super(n4 + 1)

                @pl.when(o + 2 < NOCT)
                def _(b=b, ob=ob):
                    gather_start(b, (ob + 2) % 4)

        for ob in range(2):
            last0 = base + (NOCT - 2 + ob) * OCT

            @pl.when(last0 < N_NODES)
            def _(ob=ob):
                out_copy(ob, base).wait()

    return fused_kernel(features, idx_pad, aux_pad)


def kernel(features, distsq, neighbour_indices):
    aux = _aux_weights(distsq)
    pad = N_PAD - N_NODES
    idx_pad = jnp.pad(neighbour_indices.astype(jnp.int32), ((0, pad), (0, 0)))
    aux_pad = jnp.pad(aux, ((0, pad), (0, 0)))
    return _sc_fused(features, idx_pad.reshape(-1, 128), aux_pad)
